# Initial kernel scaffold; baseline (speedup 1.0000x reference)
import jax
import jax.numpy as jnp
from jax.experimental import pallas as pl


def kernel(x, token_table, pos_table):
    raise NotImplementedError("write your pallas kernel here")


# speedup vs baseline: 2.5753x; 2.5753x over previous
import functools

import jax
import jax.numpy as jnp
from jax import lax
from jax.experimental import pallas as pl
from jax.experimental.pallas import tpu as pltpu
from jax.experimental.pallas import tpu_sc as plsc

_L = 200
_D = 64
_LANES = 16
_NVEC = _D // _LANES


@functools.lru_cache(maxsize=None)
def _build(n_rows: int, vocab: int):
    info = plsc.get_sparse_core_info()
    nw = info.num_cores * info.num_subcores
    rows_per_w = n_rows // nw
    chunk = 800
    n_chunks = rows_per_w // chunk
    sub = 80
    n_sub = chunk // sub

    mesh = plsc.VectorSubcoreMesh(core_axis_name="c", subcore_axis_name="s")

    @functools.partial(
        pl.kernel,
        mesh=mesh,
        compiler_params=pltpu.CompilerParams(use_tc_tiling_on_sc=False),
        out_type=jax.ShapeDtypeStruct((n_rows, _D), jnp.float32),
        scratch_types=[
            pltpu.VMEM((chunk,), jnp.int32),
            pltpu.VMEM((chunk, _D), jnp.float32),
            pltpu.VMEM((_L, _D), jnp.float32),
            pltpu.SemaphoreType.DMA,
        ],
    )
    def k(x_hbm, tok_hbm, pos_hbm, out_hbm, idx_v, rows_v, pos_v, sem):
        wid = lax.axis_index("s") * info.num_cores + lax.axis_index("c")
        base = wid * rows_per_w
        pltpu.sync_copy(pos_hbm, pos_v)

        def chunk_body(g, carry):
            off = base + g * chunk
            pltpu.sync_copy(x_hbm.at[pl.ds(off, chunk)], idx_v)
            for j in range(n_sub):
                pltpu.async_copy(
                    tok_hbm.at[idx_v.at[pl.ds(j * sub, sub)]],
                    rows_v.at[pl.ds(j * sub, sub)],
                    sem,
                ).wait()

            def pos_body(p, carry2):
                pv = [pos_v[p, pl.ds(kk * _LANES, _LANES)] for kk in range(_NVEC)]
                for s in range(chunk // _L):
                    r = s * _L + p
                    for kk in range(_NVEC):
                        sl = pl.ds(kk * _LANES, _LANES)
                        rows_v[r, sl] = rows_v[r, sl] + pv[kk]
                return carry2

            lax.fori_loop(0, _L, pos_body, 0)
            pltpu.sync_copy(rows_v, out_hbm.at[pl.ds(off, chunk)])
            return carry

        lax.fori_loop(0, n_chunks, chunk_body, 0)

    return k


def kernel(x, token_table, pos_table):
    b, l = x.shape
    flat = _build(b * l, token_table.shape[0])(
        x.reshape(b * l).astype(jnp.int32), token_table, pos_table
    )
    return flat.reshape(b, l, _D)


# speedup vs baseline: 3.1877x; 1.2378x over previous
import functools

import jax
import jax.numpy as jnp
from jax import lax
from jax.experimental import pallas as pl
from jax.experimental.pallas import tpu as pltpu
from jax.experimental.pallas import tpu_sc as plsc

_L = 200
_D = 64
_LANES = 16
_NVEC = _D // _LANES


@functools.lru_cache(maxsize=None)
def _build(n_rows: int, vocab: int):
    info = plsc.get_sparse_core_info()
    nw = info.num_cores * info.num_subcores
    rows_per_w = n_rows // nw
    chunk = 800
    n_chunks = rows_per_w // chunk
    sub = 80
    n_sub = chunk // sub

    mesh = plsc.VectorSubcoreMesh(core_axis_name="c", subcore_axis_name="s")

    @functools.partial(
        pl.kernel,
        mesh=mesh,
        compiler_params=pltpu.CompilerParams(use_tc_tiling_on_sc=False),
        out_type=jax.ShapeDtypeStruct((n_rows, _D), jnp.float32),
        scratch_types=[
            pltpu.VMEM((2, chunk), jnp.int32),
            pltpu.VMEM((2, chunk, _D), jnp.float32),
            pltpu.VMEM((_L, _D), jnp.float32),
            pltpu.SemaphoreType.DMA,
            pltpu.SemaphoreType.DMA,
            pltpu.SemaphoreType.DMA,
            pltpu.SemaphoreType.DMA,
    ],
    )
    def k(x_hbm, tok_hbm, pos_hbm, out_hbm, idx_v, rows_v, pos_v,
          gsem0, gsem1, ssem0, ssem1):
        gsem = (gsem0, gsem1)
        ssem = (ssem0, ssem1)
        wid = lax.axis_index("s") * info.num_cores + lax.axis_index("c")
        base = wid * rows_per_w
        pltpu.sync_copy(pos_hbm, pos_v)

        def start_gather(g, b):
            off = base + g * chunk
            pltpu.sync_copy(x_hbm.at[pl.ds(off, chunk)], idx_v.at[b])
            return [
                pltpu.async_copy(
                    tok_hbm.at[idx_v.at[b, pl.ds(j * sub, sub)]],
                    rows_v.at[b, pl.ds(j * sub, sub)],
                    gsem[b],
                )
                for j in range(n_sub)
            ]

        def add_pos(b):
            def pos_body(p, carry):
                pv = [pos_v[p, pl.ds(kk * _LANES, _LANES)] for kk in range(_NVEC)]
                for s in range(chunk // _L):
                    r = s * _L + p
                    for kk in range(_NVEC):
                        sl = pl.ds(kk * _LANES, _LANES)
                        rows_v[b, r, sl] = rows_v[b, r, sl] + pv[kk]
                return carry

            lax.fori_loop(0, _L, pos_body, 0)

        gathers = {0: start_gather(0, 0)}
        scatters = {}
        for g in range(n_chunks):
            b = g & 1
            if g + 1 < n_chunks:
                if g >= 1:
                    scatters.pop(g - 1).wait()
                gathers[g + 1] = start_gather(g + 1, b ^ 1)
            for h in gathers.pop(g):
                h.wait()
            add_pos(b)
            scatters[g] = pltpu.async_copy(
                rows_v.at[b], out_hbm.at[pl.ds(base + g * chunk, chunk)], ssem[b]
            )
        scatters.pop(n_chunks - 2).wait()
        scatters.pop(n_chunks - 1).wait()

    return k


def kernel(x, token_table, pos_table):
    b, l = x.shape
    flat = _build(b * l, token_table.shape[0])(
        x.reshape(b * l).astype(jnp.int32), token_table, pos_table
    )
    return flat.reshape(b, l, _D)


# speedup vs baseline: 3.8182x; 1.1978x over previous
import functools

import jax
import jax.numpy as jnp
from jax import lax
from jax.experimental import pallas as pl
from jax.experimental.pallas import tpu as pltpu
from jax.experimental.pallas import tpu_sc as plsc

_L = 200
_D = 64
_B = 1024
_LANES = 16
_NVEC = _D // _LANES
_BT = _B // 128
_NSLAB = _L * _BT
_PAD = 129


@functools.lru_cache(maxsize=None)
def _build(vocab: int):
    info = plsc.get_sparse_core_info()
    nw = info.num_cores * info.num_subcores
    per_w = _NSLAB // nw
    half = per_w // 2

    mesh = plsc.VectorSubcoreMesh(core_axis_name="c", subcore_axis_name="s")

    @functools.partial(
        pl.kernel,
        mesh=mesh,
        compiler_params=pltpu.CompilerParams(
            use_tc_tiling_on_sc=False, needs_layout_passes=False),
        out_type=jax.ShapeDtypeStruct((_L, _D // 8, _BT, 8, 128), jnp.float32),
        scratch_types=[
            pltpu.VMEM((per_w * 128,), jnp.int32),
            pltpu.VMEM((2, 128, _D), jnp.float32),
            pltpu.VMEM((2, _D // 8, 8, _PAD), jnp.float32),
            pltpu.VMEM((_L, _D), jnp.float32),
            pltpu.SemaphoreType.DMA,
            pltpu.SemaphoreType.DMA,
            pltpu.SemaphoreType.DMA,
            pltpu.SemaphoreType.DMA,
        ],
    )
    def k(xt_hbm, tok_hbm, pos_hbm, out_hbm, idx_all, rows_v, pout, pos_v,
          gsem0, gsem1, ssem0, ssem1):
        gsem = (gsem0, gsem1)
        ssem = (ssem0, ssem1)
        wid = lax.axis_index("s") * info.num_cores + lax.axis_index("c")
        first = wid * per_w
        pltpu.sync_copy(xt_hbm.at[pl.ds(first * 128, per_w * 128)], idx_all)
        pltpu.sync_copy(pos_hbm, pos_v)

        io = lax.iota(jnp.int32, 16)
        dt_idx = [(16 * kk + io) >> 3 for kk in range(_NVEC)]
        di_idx = [(16 * kk + io) & 7 for kk in range(_NVEC)]

        def gather_desc(buf, sl):
            return pltpu.make_async_copy(
                tok_hbm.at[idx_all.at[pl.ds(sl * 128, 128)]],
                rows_v.at[buf], gsem[buf])

        def out_desc(buf, s):
            l = s >> 3
            bt = s & 7
            return pltpu.make_async_copy(
                pout.at[buf, :, :, pl.ds(0, 128)],
                out_hbm.at[l, :, bt], ssem[buf])

        def compute(buf, s):
            l = s >> 3
            pv = [pos_v[l, pl.ds(16 * kk, 16)] for kk in range(_NVEC)]

            def tbody(t, carry):
                for u in range(4):
                    b = t * 4 + u
                    bvec = jnp.full((16,), b, dtype=jnp.int32)
                    for kk in range(_NVEC):
                        v = rows_v[buf, b, pl.ds(16 * kk, 16)] + pv[kk]
                        plsc.store_scatter(
                            pout.at[buf], [dt_idx[kk], di_idx[kk], bvec], v)
                return carry

            lax.fori_loop(0, 32, tbody, 0)

        gather_desc(0, 0).start()
        gather_desc(1, 1).start()

        def gbody(g, carry):
            for buf in range(2):
                sl = 2 * g + buf
                s = first + sl
                gather_desc(buf, sl).wait()

                @pl.when(g > 0)
                def _():
                    out_desc(buf, s).wait()

                compute(buf, s)
                out_desc(buf, s).start()

                @pl.when(g < half - 1)
                def _():
                    gather_desc(buf, sl + 2).start()
            return carry

        lax.fori_loop(0, half, gbody, 0)
        out_desc(0, first + per_w - 2).wait()
        out_desc(1, first + per_w - 1).wait()

    return k


def kernel(x, token_table, pos_table):
    b, l = x.shape
    p = _build(token_table.shape[0])(
        x.T.reshape(b * l).astype(jnp.int32), token_table, pos_table
    )
    return p.transpose(0, 1, 3, 2, 4).reshape(l, _D, b).transpose(2, 0, 1)
